# Initial kernel scaffold; baseline (speedup 1.0000x reference)
import jax
import jax.numpy as jnp
from jax.experimental import pallas as pl


def kernel(x, edge_index, edge_type, emb, W, b):
    raise NotImplementedError("write your pallas kernel here")


# speedup vs baseline: 7.3173x; 7.3173x over previous
import functools

import jax
import jax.numpy as jnp
from jax import lax
from jax.experimental import pallas as pl
from jax.experimental.pallas import tpu as pltpu
from jax.experimental.pallas import tpu_sc as plsc

N = 10000
E = 160000
F = 128
NT = 4

NC = 2
NS = 16
NW = NC * NS
CH = 128
E_PAD = 163840
NCH = E_PAD // (NW * CH)
DUMMY = N
N_ACC = 10240

BN = 1000


def _mm_body(emb_ref, w_ref, b_ref, x_ref, y_ref):
    emb = emb_ref[...]
    m = jnp.max(emb, axis=-1, keepdims=True)
    ex = jnp.exp(emb - m)
    attr = ex / jnp.sum(ex, axis=-1, keepdims=True)
    wf = w_ref[...].reshape(NT, F * F)
    mt = jnp.dot(attr, wf, preferred_element_type=jnp.float32).reshape(NT, F, F)
    bt = jnp.dot(attr, b_ref[...], preferred_element_type=jnp.float32)
    xb = x_ref[...]
    for t in range(NT):
        yt = lax.dot_general(xb, mt[t], (((1,), (1,)), ((), ())),
                             preferred_element_type=jnp.float32)
        y_ref[:, t * F:(t + 1) * F] = yt + bt[t][None, :]


_mm = pl.pallas_call(
    _mm_body,
    grid=(N // BN,),
    in_specs=[
        pl.BlockSpec((NT, NT), lambda i: (0, 0)),
        pl.BlockSpec((NT, F, F), lambda i: (0, 0, 0)),
        pl.BlockSpec((NT, F), lambda i: (0, 0)),
        pl.BlockSpec((BN, F), lambda i: (i, 0)),
    ],
    out_specs=pl.BlockSpec((BN, NT * F), lambda i: (i, 0)),
    out_shape=jax.ShapeDtypeStruct((N, NT * F), jnp.float32),
)


def _sc_body(y_hbm, g_hbm, d_hbm, z_hbm, out_hbm, gbuf, dbuf, rbuf, acc, sem):
    c = lax.axis_index("c")
    s = lax.axis_index("s")
    wid = s * NC + c
    pltpu.sync_copy(z_hbm, rbuf)
    for k in range(N_ACC // (NS * CH)):
        pltpu.sync_copy(rbuf, acc.at[pl.ds((s * (N_ACC // (NS * CH)) + k) * CH, CH)])
    pltpu.sync_copy(g_hbm.at[wid], gbuf)
    pltpu.sync_copy(d_hbm.at[wid], dbuf)
    plsc.subcore_barrier()

    def body(j, carry):
        pltpu.async_copy(y_hbm.at[gbuf.at[j]], rbuf, sem).wait()
        pltpu.sync_copy(rbuf, acc.at[dbuf.at[j]], add=True)
        return carry

    lax.fori_loop(0, NCH, body, 0)
    plsc.subcore_barrier()
    pltpu.sync_copy(acc.at[pl.ds(s * (N_ACC // NS), N_ACC // NS)],
                    out_hbm.at[c, pl.ds(s * (N_ACC // NS), N_ACC // NS)])


@functools.cache
def _get_sc():
    return functools.partial(
        pl.kernel,
        out_type=jax.ShapeDtypeStruct((NC, N_ACC, F), jnp.float32),
        mesh=plsc.VectorSubcoreMesh(core_axis_name="c", subcore_axis_name="s"),
        scratch_types=[
            pltpu.VMEM((NCH, CH), jnp.int32),
            pltpu.VMEM((NCH, CH), jnp.int32),
            pltpu.VMEM((CH, F), jnp.float32),
            pltpu.VMEM_SHARED((N_ACC, F), jnp.float32),
            pltpu.SemaphoreType.DMA,
        ],
    )(_sc_body)


def _add_body(p_ref, o_ref):
    o_ref[...] = p_ref[0] + p_ref[1]


_add = pl.pallas_call(
    _add_body,
    grid=(N // BN,),
    in_specs=[pl.BlockSpec((NC, BN, F), lambda i: (0, i, 0))],
    out_specs=pl.BlockSpec((BN, F), lambda i: (i, 0)),
    out_shape=jax.ShapeDtypeStruct((N, F), jnp.float32),
)


def kernel(x, edge_index, edge_type, emb, W, b):
    src = edge_index[0]
    dst = edge_index[1]
    y2 = _mm(emb, W, b, x).reshape(N * NT, F)
    g = src * NT + edge_type
    pad = E_PAD - E
    gp = jnp.concatenate([g, jnp.zeros((pad,), jnp.int32)])
    dp = jnp.concatenate([dst, jnp.full((pad,), DUMMY, jnp.int32)])
    g3 = gp.reshape(NW, NCH, CH)
    d3 = dp.reshape(NW, NCH, CH)
    z = jnp.zeros((CH, F), jnp.float32)
    partial = _get_sc()(y2, g3, d3, z)
    return _add(partial)


# speedup vs baseline: 8.1404x; 1.1125x over previous
import functools

import jax
import jax.numpy as jnp
from jax import lax
from jax.experimental import pallas as pl
from jax.experimental.pallas import tpu as pltpu
from jax.experimental.pallas import tpu_sc as plsc

N = 10000
E = 160000
F = 128
NT = 4

NC = 2
NS = 16
NW = NC * NS
CH = 128
E_PAD = 163840
NCH = E_PAD // (NW * CH)
DUMMY = N
N_ACC = 10240

BN = 1000


def _mm_body(emb_ref, w_ref, b_ref, x_ref, y_ref):
    emb = emb_ref[...]
    m = jnp.max(emb, axis=-1, keepdims=True)
    ex = jnp.exp(emb - m)
    attr = ex / jnp.sum(ex, axis=-1, keepdims=True)
    wf = w_ref[...].reshape(NT, F * F)
    mt = jnp.dot(attr, wf, preferred_element_type=jnp.float32).reshape(NT, F, F)
    bt = jnp.dot(attr, b_ref[...], preferred_element_type=jnp.float32)
    xb = x_ref[...]
    for t in range(NT):
        yt = lax.dot_general(xb, mt[t], (((1,), (1,)), ((), ())),
                             preferred_element_type=jnp.float32)
        y_ref[:, t * F:(t + 1) * F] = yt + bt[t][None, :]


_mm = pl.pallas_call(
    _mm_body,
    grid=(N // BN,),
    in_specs=[
        pl.BlockSpec((NT, NT), lambda i: (0, 0)),
        pl.BlockSpec((NT, F, F), lambda i: (0, 0, 0)),
        pl.BlockSpec((NT, F), lambda i: (0, 0)),
        pl.BlockSpec((BN, F), lambda i: (i, 0)),
    ],
    out_specs=pl.BlockSpec((BN, NT * F), lambda i: (i, 0)),
    out_shape=jax.ShapeDtypeStruct((N, NT * F), jnp.float32),
)


def _sc_body(y_hbm, g_hbm, d_hbm, z_hbm, out_hbm, gbuf, dbuf, rbuf0, rbuf1,
             acc, sg0, sg1, ss0, ss1):
    c = lax.axis_index("c")
    s = lax.axis_index("s")
    wid = s * NC + c
    pltpu.sync_copy(z_hbm, rbuf0)
    for k in range(N_ACC // (NS * CH)):
        pltpu.sync_copy(rbuf0, acc.at[pl.ds((s * (N_ACC // (NS * CH)) + k) * CH, CH)])
    pltpu.sync_copy(g_hbm.at[wid], gbuf)
    pltpu.sync_copy(d_hbm.at[wid], dbuf)
    plsc.subcore_barrier()

    pltpu.async_copy(y_hbm.at[gbuf.at[0]], rbuf0, sg0)
    NI = NCH // 2

    def body(i, carry):
        j0 = 2 * i

        @pl.when(i > 0)
        def _():
            pltpu.make_async_copy(rbuf1, acc.at[dbuf.at[j0]], ss1).wait()

        pltpu.async_copy(y_hbm.at[gbuf.at[j0 + 1]], rbuf1, sg1)
        pltpu.make_async_copy(y_hbm.at[gbuf.at[j0]], rbuf0, sg0).wait()
        pltpu.async_copy(rbuf0, acc.at[dbuf.at[j0]], ss0, add=True)
        pltpu.make_async_copy(rbuf0, acc.at[dbuf.at[j0]], ss0).wait()

        @pl.when(i < NI - 1)
        def _():
            pltpu.async_copy(y_hbm.at[gbuf.at[j0 + 2]], rbuf0, sg0)

        pltpu.make_async_copy(y_hbm.at[gbuf.at[j0 + 1]], rbuf1, sg1).wait()
        pltpu.async_copy(rbuf1, acc.at[dbuf.at[j0 + 1]], ss1, add=True)
        return carry

    lax.fori_loop(0, NI, body, 0)
    pltpu.make_async_copy(rbuf1, acc.at[dbuf.at[NCH - 1]], ss1).wait()
    plsc.subcore_barrier()
    pltpu.sync_copy(acc.at[pl.ds(s * (N_ACC // NS), N_ACC // NS)],
                    out_hbm.at[c, pl.ds(s * (N_ACC // NS), N_ACC // NS)])


@functools.cache
def _get_sc():
    return functools.partial(
        pl.kernel,
        out_type=jax.ShapeDtypeStruct((NC, N_ACC, F), jnp.float32),
        mesh=plsc.VectorSubcoreMesh(core_axis_name="c", subcore_axis_name="s"),
        scratch_types=[
            pltpu.VMEM((NCH, CH), jnp.int32),
            pltpu.VMEM((NCH, CH), jnp.int32),
            pltpu.VMEM((CH, F), jnp.float32),
            pltpu.VMEM((CH, F), jnp.float32),
            pltpu.VMEM_SHARED((N_ACC, F), jnp.float32),
            pltpu.SemaphoreType.DMA,
            pltpu.SemaphoreType.DMA,
            pltpu.SemaphoreType.DMA,
            pltpu.SemaphoreType.DMA,
        ],
    )(_sc_body)


def _add_body(p_ref, o_ref):
    o_ref[...] = p_ref[0] + p_ref[1]


_add = pl.pallas_call(
    _add_body,
    grid=(N // BN,),
    in_specs=[pl.BlockSpec((NC, BN, F), lambda i: (0, i, 0))],
    out_specs=pl.BlockSpec((BN, F), lambda i: (i, 0)),
    out_shape=jax.ShapeDtypeStruct((N, F), jnp.float32),
)


def kernel(x, edge_index, edge_type, emb, W, b):
    src = edge_index[0]
    dst = edge_index[1]
    y2 = _mm(emb, W, b, x).reshape(N * NT, F)
    g = src * NT + edge_type
    pad = E_PAD - E
    gp = jnp.concatenate([g, jnp.zeros((pad,), jnp.int32)])
    dp = jnp.concatenate([dst, jnp.full((pad,), DUMMY, jnp.int32)])
    g3 = gp.reshape(NW, NCH, CH)
    d3 = dp.reshape(NW, NCH, CH)
    z = jnp.zeros((CH, F), jnp.float32)
    partial = _get_sc()(y2, g3, d3, z)
    return _add(partial)


# speedup vs baseline: 17.3125x; 2.1267x over previous
import functools

import jax
import jax.numpy as jnp
from jax import lax
from jax.experimental import pallas as pl
from jax.experimental.pallas import tpu as pltpu
from jax.experimental.pallas import tpu_sc as plsc

N = 10000
E = 160000
F = 128
NT = 4

NC = 2
NS = 16
NW = NC * NS
CH = 128
E_PAD = 163840
NCH = E_PAD // (NW * CH)
N_ACC = 10240

BN = 1000


def _mm_body(emb_ref, w_ref, b_ref, x_ref, y_ref):
    @pl.when(pl.program_id(0) == N // BN)
    def _():
        y_ref[...] = jnp.zeros_like(y_ref)

    @pl.when(pl.program_id(0) < N // BN)
    def _():
        emb = emb_ref[...]
        m = jnp.max(emb, axis=-1, keepdims=True)
        ex = jnp.exp(emb - m)
        attr = ex / jnp.sum(ex, axis=-1, keepdims=True)
        wf = w_ref[...].reshape(NT, F * F)
        mt = jnp.dot(attr, wf, preferred_element_type=jnp.float32).reshape(NT, F, F)
        bt = jnp.dot(attr, b_ref[...], preferred_element_type=jnp.float32)
        xb = x_ref[...]
        for t in range(NT):
            yt = lax.dot_general(xb, mt[t], (((1,), (1,)), ((), ())),
                                 preferred_element_type=jnp.float32)
            y_ref[:, t * F:(t + 1) * F] = yt + bt[t][None, :]


_mm = pl.pallas_call(
    _mm_body,
    grid=(N // BN + 1,),
    in_specs=[
        pl.BlockSpec((NT, NT), lambda i: (0, 0)),
        pl.BlockSpec((NT, F, F), lambda i: (0, 0, 0)),
        pl.BlockSpec((NT, F), lambda i: (0, 0)),
        pl.BlockSpec((BN, F), lambda i: (jnp.minimum(i, N // BN - 1), 0)),
    ],
    out_specs=pl.BlockSpec((BN, NT * F), lambda i: (i, 0)),
    out_shape=jax.ShapeDtypeStruct((N + BN, NT * F), jnp.float32),
)


def _sc_body(y_hbm, g_hbm, d_hbm, z_hbm, out_hbm, gbuf, dbuf, rbuf0, rbuf1,
             acc, sg0, sg1, ss0, ss1):
    c = lax.axis_index("c")
    s = lax.axis_index("s")
    wid = s * NC + c
    pltpu.sync_copy(z_hbm, rbuf0)
    nz = N_ACC // (NS * CH)
    for k in range(nz):
        pltpu.async_copy(rbuf0, acc.at[pl.ds((s * nz + k) * CH, CH)], ss0)
    pltpu.sync_copy(g_hbm.at[wid], gbuf)
    pltpu.sync_copy(d_hbm.at[wid], dbuf)
    for k in range(nz):
        pltpu.make_async_copy(rbuf0, acc.at[pl.ds((s * nz + k) * CH, CH)], ss0).wait()
    plsc.subcore_barrier()

    pltpu.async_copy(y_hbm.at[gbuf.at[0]], rbuf0, sg0)
    NI = NCH // 2

    def body(i, carry):
        j0 = 2 * i

        @pl.when(i > 0)
        def _():
            pltpu.make_async_copy(rbuf1, acc.at[dbuf.at[j0]], ss1).wait()

        pltpu.async_copy(y_hbm.at[gbuf.at[j0 + 1]], rbuf1, sg1)
        pltpu.make_async_copy(y_hbm.at[gbuf.at[j0]], rbuf0, sg0).wait()
        pltpu.async_copy(rbuf0, acc.at[dbuf.at[j0]], ss0, add=True)
        pltpu.make_async_copy(rbuf0, acc.at[dbuf.at[j0]], ss0).wait()

        @pl.when(i < NI - 1)
        def _():
            pltpu.async_copy(y_hbm.at[gbuf.at[j0 + 2]], rbuf0, sg0)

        pltpu.make_async_copy(y_hbm.at[gbuf.at[j0 + 1]], rbuf1, sg1).wait()
        pltpu.async_copy(rbuf1, acc.at[dbuf.at[j0 + 1]], ss1, add=True)
        return carry

    lax.fori_loop(0, NI, body, 0)
    pltpu.make_async_copy(rbuf1, acc.at[dbuf.at[NCH - 1]], ss1).wait()
    plsc.subcore_barrier()
    pltpu.sync_copy(acc.at[pl.ds(s * (N_ACC // NS), N_ACC // NS)],
                    out_hbm.at[c, pl.ds(s * (N_ACC // NS), N_ACC // NS)])


@functools.cache
def _get_sc():
    return functools.partial(
        pl.kernel,
        out_type=jax.ShapeDtypeStruct((NC, N_ACC, F), jnp.float32),
        mesh=plsc.VectorSubcoreMesh(core_axis_name="c", subcore_axis_name="s"),
        scratch_types=[
            pltpu.VMEM((NCH, CH), jnp.int32),
            pltpu.VMEM((NCH, CH), jnp.int32),
            pltpu.VMEM((CH, F), jnp.float32),
            pltpu.VMEM((CH, F), jnp.float32),
            pltpu.VMEM_SHARED((N_ACC, F), jnp.float32),
            pltpu.SemaphoreType.DMA,
            pltpu.SemaphoreType.DMA,
            pltpu.SemaphoreType.DMA,
            pltpu.SemaphoreType.DMA,
        ],
    )(_sc_body)


def _add_body(p_ref, o_ref):
    o_ref[...] = p_ref[0] + p_ref[1]


_add = pl.pallas_call(
    _add_body,
    grid=(N // BN,),
    in_specs=[pl.BlockSpec((NC, BN, F), lambda i: (0, i, 0))],
    out_specs=pl.BlockSpec((BN, F), lambda i: (i, 0)),
    out_shape=jax.ShapeDtypeStruct((N, F), jnp.float32),
)


def kernel(x, edge_index, edge_type, emb, W, b):
    src = edge_index[0]
    dst = edge_index[1]
    y2 = _mm(emb, W, b, x).reshape((N + BN) * NT, F)
    g = src * NT + edge_type
    pad = E_PAD - E
    ar = jnp.arange(pad, dtype=jnp.int32)
    gp = jnp.concatenate([g, N * NT + ar % (BN * NT)])
    dp = jnp.concatenate([dst, ar % N])
    g3 = gp.reshape(NW, NCH, CH)
    d3 = dp.reshape(NW, NCH, CH)
    z = jnp.zeros((CH, F), jnp.float32)
    partial = _get_sc()(y2, g3, d3, z)
    return _add(partial)
